# baseline (speedup 1.0000x reference)
import jax
from jax.experimental import pallas as pl
from jax.experimental.pallas import tpu as pltpu


_N_CHUNKS = 16
_ROWS = 100000
_CHUNK = _ROWS // _N_CHUNKS


def _identity_copy_kernel(x_ref, o_ref):
    def body(sems):
        copies = []
        for i in range(_N_CHUNKS):
            cp = pltpu.make_async_copy(
                x_ref.at[pl.ds(i * _CHUNK, _CHUNK), :],
                o_ref.at[pl.ds(i * _CHUNK, _CHUNK), :],
                sems.at[i],
            )
            cp.start()
            copies.append(cp)
        for cp in copies:
            cp.wait()

    pl.run_scoped(body, pltpu.SemaphoreType.DMA((_N_CHUNKS,)))


def kernel(x):
    return pl.pallas_call(
        _identity_copy_kernel,
        in_specs=[pl.BlockSpec(memory_space=pl.ANY)],
        out_specs=pl.BlockSpec(memory_space=pl.ANY),
        out_shape=jax.ShapeDtypeStruct(x.shape, x.dtype),
    )(x)


# speedup vs baseline: 30.6868x; 30.6868x over previous
import jax
from jax.experimental import pallas as pl
from jax.experimental.pallas import tpu as pltpu


_BLOCK_ROWS = 2000


def _identity_copy_kernel(x_ref, o_ref):
    o_ref[...] = x_ref[...]


def kernel(x):
    rows = x.shape[0]
    return pl.pallas_call(
        _identity_copy_kernel,
        grid=(rows // _BLOCK_ROWS,),
        in_specs=[pl.BlockSpec((_BLOCK_ROWS, x.shape[1]), lambda i: (i, 0))],
        out_specs=pl.BlockSpec((_BLOCK_ROWS, x.shape[1]), lambda i: (i, 0)),
        out_shape=jax.ShapeDtypeStruct(x.shape, x.dtype),
        compiler_params=pltpu.CompilerParams(
            dimension_semantics=("arbitrary",),
        ),
    )(x)
